# baseline (speedup 1.0000x reference)
import functools

import jax
import jax.numpy as jnp
from jax import lax
from jax.experimental import pallas as pl
from jax.experimental.pallas import tpu as pltpu
from jax.experimental.pallas import tpu_sc as plsc

N = 4096
M = 16384
C_IN = 256
C_SKIP = 128
K = 3
C2 = 256
GROUPS = 32
GSIZE = C2 // GROUPS
EPS = 1e-5

BM = 1024
BM2 = 2048

_HI = jax.lax.Precision.HIGHEST
_DEF = jax.lax.Precision.DEFAULT


def _dot(a, b, precision=_HI):
    return jax.lax.dot_general(a, b, (((1,), (0,)), ((), ())),
                               precision=precision,
                               preferred_element_type=jnp.float32)


def _top3_body(aq_ref, asT_ref, sn_ref, idx_ref, w_ref):
    aq = aq_ref[...]
    ncross2 = _dot(aq, asT_ref[...], precision=_DEF)
    qn = aq[:, 3:4]
    d2 = jnp.maximum((qn + sn_ref[...]) + ncross2, 0.0)

    CW = 128
    NCH = N // CW
    BIG = jnp.float32(3.0e38)
    lane = lax.broadcasted_iota(jnp.int32, (BM, CW), 1).astype(jnp.float32)
    m1 = jnp.full((BM, CW), BIG)
    m2 = jnp.full((BM, CW), BIG)
    m3 = jnp.full((BM, CW), BIG)
    i1 = jnp.zeros((BM, CW), jnp.float32)
    i2 = jnp.zeros((BM, CW), jnp.float32)
    i3 = jnp.zeros((BM, CW), jnp.float32)
    for c in range(NCH):
        v = d2[:, c * CW:(c + 1) * CW]
        iv = lane + jnp.float32(c * CW)
        c1 = v < m1
        c2 = v < m2
        c3 = v < m3
        m3 = jnp.where(c2, m2, jnp.where(c3, v, m3))
        i3 = jnp.where(c2, i2, jnp.where(c3, iv, i3))
        m2 = jnp.where(c1, m1, jnp.where(c2, v, m2))
        i2 = jnp.where(c1, i1, jnp.where(c2, iv, i2))
        m1 = jnp.where(c1, v, m1)
        i1 = jnp.where(c1, iv, i1)

    cand = jnp.concatenate([m1, m2, m3], axis=1)
    icand = jnp.concatenate([i1, i2, i3], axis=1)
    idxs = []
    ws = []
    for j in range(K):
        vmin = jnp.min(cand, axis=1, keepdims=True)
        idxj = jnp.min(jnp.where(cand == vmin, icand, float(N)),
                       axis=1, keepdims=True)
        idxs.append(idxj)
        ws.append(1.0 / jnp.maximum(vmin, 1e-16))
        if j < K - 1:
            cand = jnp.where(icand == idxj, BIG, cand)
    wsum = ws[0] + ws[1] + ws[2]
    zf = jnp.zeros((BM, 1), jnp.float32)
    idx_ref[...] = jnp.concatenate(idxs + [zf] * 5, axis=1).astype(jnp.int32)
    w_ref[...] = jnp.concatenate([wj / wsum for wj in ws] + [zf] * 5, axis=1)


def _make_top3(mh, off):
    boff = off // BM

    @jax.jit
    def f(aq, asT, sn):
        return pl.pallas_call(
            _top3_body,
            grid=(mh // BM,),
            in_specs=[
                pl.BlockSpec((BM, 8), lambda i: (i + boff, 0)),
                pl.BlockSpec((8, N), lambda i: (0, 0)),
                pl.BlockSpec((1, N), lambda i: (0, 0)),
            ],
            out_specs=[pl.BlockSpec((BM, 8), lambda i: (i, 0)),
                       pl.BlockSpec((BM, 8), lambda i: (i, 0))],
            out_shape=[jax.ShapeDtypeStruct((mh, 8), jnp.int32),
                       jax.ShapeDtypeStruct((mh, 8), jnp.float32)],
        )(aq, asT, sn)

    return f


_SC_CH = 128


def _gather3(x, idx0, idx1, idx2):
    mh = idx0.shape[0]
    info = plsc.get_sparse_core_info()
    nw = info.num_cores * info.num_subcores
    b_per_w = mh // nw
    n_ch = b_per_w // _SC_CH
    mesh = plsc.VectorSubcoreMesh(core_axis_name="c", subcore_axis_name="s")

    @functools.partial(
        pl.kernel, mesh=mesh,
        out_type=[jax.ShapeDtypeStruct((mh, C_IN), jnp.float32)] * K,
        scratch_types=[
            pltpu.VMEM((b_per_w,), jnp.int32),
            pltpu.VMEM((b_per_w,), jnp.int32),
            pltpu.VMEM((b_per_w,), jnp.int32),
            pltpu.VMEM((_SC_CH, C_IN), jnp.float32),
            pltpu.VMEM((_SC_CH, C_IN), jnp.float32),
            pltpu.SemaphoreType.DMA,
            pltpu.SemaphoreType.DMA,
            pltpu.SemaphoreType.DMA,
            pltpu.SemaphoreType.DMA,
        ],
    )
    def k(x_hbm, i0_hbm, i1_hbm, i2_hbm, g0_hbm, g1_hbm, g2_hbm,
          i0_v, i1_v, i2_v, rows_a, rows_b, sga, sgb, ssa, ssb):
        wid = lax.axis_index("s") * info.num_cores + lax.axis_index("c")
        wbase = wid * b_per_w
        pltpu.sync_copy(i0_hbm.at[pl.ds(wbase, b_per_w)], i0_v)
        pltpu.sync_copy(i1_hbm.at[pl.ds(wbase, b_per_w)], i1_v)
        pltpu.sync_copy(i2_hbm.at[pl.ds(wbase, b_per_w)], i2_v)
        idxs = (i0_v, i1_v, i2_v)
        outs = (g0_hbm, g1_hbm, g2_hbm)
        steps = [(j, c) for j in range(K) for c in range(n_ch)]
        bufs = (rows_a, rows_b)
        gsems = (sga, sgb)
        ssems = (ssa, ssb)
        nst = len(steps)
        gh = [None] * nst
        sh = [None] * nst

        def scatter(t):
            j, c = steps[t]
            gh[t].wait()
            sh[t] = pltpu.async_copy(
                bufs[t % 2], outs[j].at[pl.ds(wbase + c * _SC_CH, _SC_CH)],
                ssems[t % 2])

        for t, (j, c) in enumerate(steps):
            if t >= 2:
                sh[t - 2].wait()
            gh[t] = pltpu.async_copy(
                x_hbm.at[idxs[j].at[pl.ds(c * _SC_CH, _SC_CH)]],
                bufs[t % 2], gsems[t % 2])
            if t >= 1:
                scatter(t - 1)
        scatter(nst - 1)
        sh[nst - 2].wait()
        sh[nst - 1].wait()

    return jax.jit(k)(x, idx0, idx1, idx2)


def _mlp_body(g0_ref, g1_ref, g2_ref, w_ref, xs_ref, w1a_ref, w1b_ref,
              w2_ref, b1_ref, b2_ref, gam_ref, bet_ref, gm_ref, gmT_ref,
              out_ref):
    w = w_ref[...]
    interp = (w[:, 0:1] * g0_ref[...] + w[:, 1:2] * g1_ref[...]
              + w[:, 2:3] * g2_ref[...])

    h = (_dot(interp, w1a_ref[...], precision=_DEF)
         + _dot(xs_ref[...], w1b_ref[...], precision=_DEF))
    h = jnp.maximum(h + b1_ref[...], 0.0)
    h = jnp.maximum(_dot(h, w2_ref[...], precision=_DEF) + b2_ref[...], 0.0)

    mean = _dot(h, gm_ref[...]) * (1.0 / GSIZE)
    msq = _dot(h * h, gm_ref[...]) * (1.0 / GSIZE)
    var = msq - mean * mean
    inv = jax.lax.rsqrt(var + EPS)
    hn = (h - _dot(mean, gmT_ref[...])) * _dot(inv, gmT_ref[...])
    out_ref[...] = hn * gam_ref[...] + bet_ref[...]


def _make_mlp(mh, off):
    boff = off // BM2

    @jax.jit
    def f(g0, g1, g2, w8, x_skip, w1a, w1b, W2, b1, b2, gamma, beta, gm, gmT):
        full = lambda r, c: pl.BlockSpec((r, c), lambda i: (0, 0))
        return pl.pallas_call(
            _mlp_body,
            grid=(mh // BM2,),
            in_specs=[
                pl.BlockSpec((BM2, C_IN), lambda i: (i, 0)),
                pl.BlockSpec((BM2, C_IN), lambda i: (i, 0)),
                pl.BlockSpec((BM2, C_IN), lambda i: (i, 0)),
                pl.BlockSpec((BM2, 8), lambda i: (i, 0)),
                pl.BlockSpec((BM2, C_SKIP), lambda i: (i + boff, 0)),
                full(C_IN, C2), full(C_SKIP, C2), full(C2, C2),
                full(1, C2), full(1, C2), full(1, C2), full(1, C2),
                full(C2, GROUPS), full(GROUPS, C2),
            ],
            out_specs=pl.BlockSpec((BM2, C2), lambda i: (i, 0)),
            out_shape=jax.ShapeDtypeStruct((mh, C2), jnp.float32),
        )(g0, g1, g2, w8, x_skip, w1a, w1b, W2, b1, b2, gamma, beta, gm, gmT)

    return f


def kernel(x, pos, reflectance, batch, x_skip, pos_skip, reflectance_skip,
           batch_skip, W1, b1, W2, b2, gamma, beta):
    qn = jnp.sum(pos_skip ** 2, axis=-1, keepdims=True)
    sn = jnp.sum(pos ** 2, axis=-1)[None, :]
    aq = jnp.concatenate([-2.0 * pos_skip, qn,
                          jnp.zeros((M, 4), jnp.float32)], axis=1)
    asT = jnp.concatenate([pos, jnp.zeros((N, 5), jnp.float32)], axis=1).T

    w1a = W1[:C_IN]
    w1b = W1[C_IN:]
    gidx = jnp.arange(C2, dtype=jnp.int32) // GSIZE
    gm = (gidx[:, None] == jnp.arange(GROUPS, dtype=jnp.int32)[None, :]).astype(jnp.float32)
    gmT = gm.T
    b1r, b2r = b1.reshape(1, C2), b2.reshape(1, C2)
    gr, br = gamma.reshape(1, C2), beta.reshape(1, C2)

    H = M // 2
    halves = []
    for off in (0, H):
        idx8, w8 = _make_top3(H, off)(aq, asT, sn)
        g0, g1, g2 = _gather3(x, idx8[:, 0], idx8[:, 1], idx8[:, 2])
        halves.append(_make_mlp(H, off)(
            g0, g1, g2, w8, x_skip, w1a, w1b, W2, b1r, b2r, gr, br, gm, gmT))
    h = jnp.concatenate(halves, axis=0)
    return (h, pos_skip, reflectance_skip, batch_skip)


# speedup vs baseline: 1.1329x; 1.1329x over previous
import functools

import jax
import jax.numpy as jnp
from jax import lax
from jax.experimental import pallas as pl
from jax.experimental.pallas import tpu as pltpu
from jax.experimental.pallas import tpu_sc as plsc

N = 4096
M = 16384
C_IN = 256
C_SKIP = 128
K = 3
C2 = 256
GROUPS = 32
GSIZE = C2 // GROUPS
EPS = 1e-5

BM = 1024
BM2 = 2048

_HI = jax.lax.Precision.HIGHEST
_DEF = jax.lax.Precision.DEFAULT


def _dot(a, b, precision=_HI):
    return jax.lax.dot_general(a, b, (((1,), (0,)), ((), ())),
                               precision=precision,
                               preferred_element_type=jnp.float32)


def _dot2(a, b):
    ahi = a.astype(jnp.bfloat16).astype(jnp.float32)
    alo = a - ahi
    return _dot(ahi, b, precision=_DEF) + _dot(alo, b, precision=_DEF)


def _top3_body(aq_ref, asT_ref, sn_ref, idx_ref, w_ref):
    aq = aq_ref[...]
    ncross2 = _dot(aq, asT_ref[...], precision=_DEF)
    qn = aq[:, 3:4]
    d2 = jnp.maximum((qn + sn_ref[...]) + ncross2, 0.0)

    CW = 128
    NCH = N // CW
    BIG = jnp.float32(3.0e38)
    lane = lax.broadcasted_iota(jnp.int32, (BM, CW), 1).astype(jnp.float32)
    m1 = jnp.full((BM, CW), BIG)
    m2 = jnp.full((BM, CW), BIG)
    m3 = jnp.full((BM, CW), BIG)
    i1 = jnp.zeros((BM, CW), jnp.float32)
    i2 = jnp.zeros((BM, CW), jnp.float32)
    i3 = jnp.zeros((BM, CW), jnp.float32)
    for c in range(NCH):
        v = d2[:, c * CW:(c + 1) * CW]
        iv = lane + jnp.float32(c * CW)
        c1 = v < m1
        c2 = v < m2
        c3 = v < m3
        m3 = jnp.where(c2, m2, jnp.where(c3, v, m3))
        i3 = jnp.where(c2, i2, jnp.where(c3, iv, i3))
        m2 = jnp.where(c1, m1, jnp.where(c2, v, m2))
        i2 = jnp.where(c1, i1, jnp.where(c2, iv, i2))
        m1 = jnp.where(c1, v, m1)
        i1 = jnp.where(c1, iv, i1)

    cand = jnp.concatenate([m1, m2, m3], axis=1)
    icand = jnp.concatenate([i1, i2, i3], axis=1)
    idxs = []
    ws = []
    for j in range(K):
        vmin = jnp.min(cand, axis=1, keepdims=True)
        idxj = jnp.min(jnp.where(cand == vmin, icand, float(N)),
                       axis=1, keepdims=True)
        idxs.append(idxj)
        ws.append(1.0 / jnp.maximum(vmin, 1e-16))
        if j < K - 1:
            cand = jnp.where(icand == idxj, BIG, cand)
    wsum = ws[0] + ws[1] + ws[2]
    zf = jnp.zeros((BM, 1), jnp.float32)
    idx_ref[...] = jnp.concatenate(idxs + [zf] * 5, axis=1).astype(jnp.int32)
    w_ref[...] = jnp.concatenate([wj / wsum for wj in ws] + [zf] * 5, axis=1)


@jax.jit
def _top3(aq, asT, sn):
    return pl.pallas_call(
        _top3_body,
        grid=(M // BM,),
        in_specs=[
            pl.BlockSpec((BM, 8), lambda i: (i, 0)),
            pl.BlockSpec((8, N), lambda i: (0, 0)),
            pl.BlockSpec((1, N), lambda i: (0, 0)),
        ],
        out_specs=[pl.BlockSpec((BM, 8), lambda i: (i, 0)),
                   pl.BlockSpec((BM, 8), lambda i: (i, 0))],
        out_shape=[jax.ShapeDtypeStruct((M, 8), jnp.int32),
                   jax.ShapeDtypeStruct((M, 8), jnp.float32)],
    )(aq, asT, sn)


_SC_CH = 128


def _gather3(x, idx0, idx1, idx2):
    info = plsc.get_sparse_core_info()
    nw = info.num_cores * info.num_subcores
    b_per_w = M // nw
    n_ch = b_per_w // _SC_CH
    mesh = plsc.VectorSubcoreMesh(core_axis_name="c", subcore_axis_name="s")

    @functools.partial(
        pl.kernel, mesh=mesh,
        out_type=[jax.ShapeDtypeStruct((M, C_IN), jnp.float32)] * K,
        scratch_types=[
            pltpu.VMEM((b_per_w,), jnp.int32),
            pltpu.VMEM((b_per_w,), jnp.int32),
            pltpu.VMEM((b_per_w,), jnp.int32),
            pltpu.VMEM((_SC_CH, C_IN), jnp.float32),
            pltpu.VMEM((_SC_CH, C_IN), jnp.float32),
            pltpu.SemaphoreType.DMA,
            pltpu.SemaphoreType.DMA,
            pltpu.SemaphoreType.DMA,
            pltpu.SemaphoreType.DMA,
        ],
    )
    def k(x_hbm, i0_hbm, i1_hbm, i2_hbm, g0_hbm, g1_hbm, g2_hbm,
          i0_v, i1_v, i2_v, rows_a, rows_b, sga, sgb, ssa, ssb):
        wid = lax.axis_index("s") * info.num_cores + lax.axis_index("c")
        wbase = wid * b_per_w
        pltpu.sync_copy(i0_hbm.at[pl.ds(wbase, b_per_w)], i0_v)
        pltpu.sync_copy(i1_hbm.at[pl.ds(wbase, b_per_w)], i1_v)
        pltpu.sync_copy(i2_hbm.at[pl.ds(wbase, b_per_w)], i2_v)
        idxs = (i0_v, i1_v, i2_v)
        outs = (g0_hbm, g1_hbm, g2_hbm)
        steps = [(j, c) for j in range(K) for c in range(n_ch)]
        bufs = (rows_a, rows_b)
        gsems = (sga, sgb)
        ssems = (ssa, ssb)
        nst = len(steps)
        gh = [None] * nst
        sh = [None] * nst

        def scatter(t):
            j, c = steps[t]
            gh[t].wait()
            sh[t] = pltpu.async_copy(
                bufs[t % 2], outs[j].at[pl.ds(wbase + c * _SC_CH, _SC_CH)],
                ssems[t % 2])

        for t, (j, c) in enumerate(steps):
            if t >= 2:
                sh[t - 2].wait()
            gh[t] = pltpu.async_copy(
                x_hbm.at[idxs[j].at[pl.ds(c * _SC_CH, _SC_CH)]],
                bufs[t % 2], gsems[t % 2])
            if t >= 1:
                scatter(t - 1)
        scatter(nst - 1)
        sh[nst - 2].wait()
        sh[nst - 1].wait()

    return jax.jit(k)(x, idx0, idx1, idx2)


def _mlp_body(g0_ref, g1_ref, g2_ref, w_ref, xs_ref, w1a_ref, w1b_ref,
              w2_ref, b1_ref, b2_ref, gam_ref, bet_ref, gm_ref, gmT_ref,
              out_ref):
    w = w_ref[...]
    interp = (w[:, 0:1] * g0_ref[...] + w[:, 1:2] * g1_ref[...]
              + w[:, 2:3] * g2_ref[...])

    h = (_dot(interp, w1a_ref[...], precision=_DEF)
         + _dot(xs_ref[...], w1b_ref[...], precision=_DEF))
    h = jnp.maximum(h + b1_ref[...], 0.0)
    h = jnp.maximum(_dot(h, w2_ref[...], precision=_DEF) + b2_ref[...], 0.0)

    gm = gm_ref[...]
    gmT = gmT_ref[...]
    mean = _dot2(h, gm) * (1.0 / GSIZE)
    msq = _dot2(h * h, gm) * (1.0 / GSIZE)
    var = msq - mean * mean
    inv = jax.lax.rsqrt(var + EPS)
    hn = (h - _dot2(mean, gmT)) * _dot2(inv, gmT)
    out_ref[...] = hn * gam_ref[...] + bet_ref[...]


@jax.jit
def _mlp(g0, g1, g2, w8, x_skip, w1a, w1b, W2, b1, b2, gamma, beta, gm, gmT):
    full = lambda r, c: pl.BlockSpec((r, c), lambda i: (0, 0))
    return pl.pallas_call(
        _mlp_body,
        grid=(M // BM2,),
        in_specs=[
            pl.BlockSpec((BM2, C_IN), lambda i: (i, 0)),
            pl.BlockSpec((BM2, C_IN), lambda i: (i, 0)),
            pl.BlockSpec((BM2, C_IN), lambda i: (i, 0)),
            pl.BlockSpec((BM2, 8), lambda i: (i, 0)),
            pl.BlockSpec((BM2, C_SKIP), lambda i: (i, 0)),
            full(C_IN, C2), full(C_SKIP, C2), full(C2, C2),
            full(1, C2), full(1, C2), full(1, C2), full(1, C2),
            full(C2, GROUPS), full(GROUPS, C2),
        ],
        out_specs=pl.BlockSpec((BM2, C2), lambda i: (i, 0)),
        out_shape=jax.ShapeDtypeStruct((M, C2), jnp.float32),
    )(g0, g1, g2, w8, x_skip, w1a, w1b, W2, b1, b2, gamma, beta, gm, gmT)


def kernel(x, pos, reflectance, batch, x_skip, pos_skip, reflectance_skip,
           batch_skip, W1, b1, W2, b2, gamma, beta):
    qn = jnp.sum(pos_skip ** 2, axis=-1, keepdims=True)
    sn = jnp.sum(pos ** 2, axis=-1)[None, :]
    aq = jnp.concatenate([-2.0 * pos_skip, qn,
                          jnp.zeros((M, 4), jnp.float32)], axis=1)
    asT = jnp.concatenate([pos, jnp.zeros((N, 5), jnp.float32)], axis=1).T

    idx8, w8 = _top3(aq, asT, sn)
    g0, g1, g2 = _gather3(x, idx8[:, 0], idx8[:, 1], idx8[:, 2])

    w1a = W1[:C_IN]
    w1b = W1[C_IN:]
    gidx = jnp.arange(C2, dtype=jnp.int32) // GSIZE
    gm = (gidx[:, None] == jnp.arange(GROUPS, dtype=jnp.int32)[None, :]).astype(jnp.float32)
    gmT = gm.T

    h = _mlp(g0, g1, g2, w8, x_skip, w1a, w1b, W2,
             b1.reshape(1, C2), b2.reshape(1, C2),
             gamma.reshape(1, C2), beta.reshape(1, C2), gm, gmT)
    return (h, pos_skip, reflectance_skip, batch_skip)


# speedup vs baseline: 1.1379x; 1.0045x over previous
import functools

import jax
import jax.numpy as jnp
from jax import lax
from jax.experimental import pallas as pl
from jax.experimental.pallas import tpu as pltpu
from jax.experimental.pallas import tpu_sc as plsc

N = 4096
M = 16384
C_IN = 256
C_SKIP = 128
K = 3
C2 = 256
GROUPS = 32
GSIZE = C2 // GROUPS
EPS = 1e-5

BM = 1024
BM2 = 2048

_HI = jax.lax.Precision.HIGHEST
_DEF = jax.lax.Precision.DEFAULT


def _dot(a, b, precision=_HI):
    return jax.lax.dot_general(a, b, (((1,), (0,)), ((), ())),
                               precision=precision,
                               preferred_element_type=jnp.float32)


def _dot2(a, b):
    ahi = a.astype(jnp.bfloat16).astype(jnp.float32)
    alo = a - ahi
    return _dot(ahi, b, precision=_DEF) + _dot(alo, b, precision=_DEF)


def _top3_body(aq_ref, asT_ref, sn_ref, i0_ref, i1_ref, i2_ref, w_ref):
    idx_refs = (i0_ref, i1_ref, i2_ref)
    aq = aq_ref[...]
    ncross2 = _dot(aq, asT_ref[...], precision=_DEF)
    qn = aq[:, 3:4]
    d2 = jnp.maximum((qn + sn_ref[...]) + ncross2, 0.0)

    CW = 128
    NCH = N // CW
    BIG = jnp.float32(3.0e38)
    lane = lax.broadcasted_iota(jnp.int32, (BM, CW), 1).astype(jnp.float32)
    m1 = jnp.full((BM, CW), BIG)
    m2 = jnp.full((BM, CW), BIG)
    m3 = jnp.full((BM, CW), BIG)
    i1 = jnp.zeros((BM, CW), jnp.float32)
    i2 = jnp.zeros((BM, CW), jnp.float32)
    i3 = jnp.zeros((BM, CW), jnp.float32)
    for c in range(NCH):
        v = d2[:, c * CW:(c + 1) * CW]
        iv = lane + jnp.float32(c * CW)
        c1 = v < m1
        c2 = v < m2
        c3 = v < m3
        m3 = jnp.where(c2, m2, jnp.where(c3, v, m3))
        i3 = jnp.where(c2, i2, jnp.where(c3, iv, i3))
        m2 = jnp.where(c1, m1, jnp.where(c2, v, m2))
        i2 = jnp.where(c1, i1, jnp.where(c2, iv, i2))
        m1 = jnp.where(c1, v, m1)
        i1 = jnp.where(c1, iv, i1)

    cand = jnp.concatenate([m1, m2, m3], axis=1)
    icand = jnp.concatenate([i1, i2, i3], axis=1)
    idxs = []
    ws = []
    for j in range(K):
        vmin = jnp.min(cand, axis=1, keepdims=True)
        idxj = jnp.min(jnp.where(cand == vmin, icand, float(N)),
                       axis=1, keepdims=True)
        idxs.append(idxj)
        ws.append(1.0 / jnp.maximum(vmin, 1e-16))
        if j < K - 1:
            cand = jnp.where(icand == idxj, BIG, cand)
    wsum = ws[0] + ws[1] + ws[2]
    for j in range(K):
        idx_refs[j][...] = idxs[j].astype(jnp.int32)
    zf = jnp.zeros((BM, 1), jnp.float32)
    w_ref[...] = jnp.concatenate([wj / wsum for wj in ws] + [zf] * 5, axis=1)


@jax.jit
def _top3(aq, asT, sn):
    return pl.pallas_call(
        _top3_body,
        grid=(M // BM,),
        in_specs=[
            pl.BlockSpec((BM, 8), lambda i: (i, 0)),
            pl.BlockSpec((8, N), lambda i: (0, 0)),
            pl.BlockSpec((1, N), lambda i: (0, 0)),
        ],
        out_specs=[pl.BlockSpec((BM, 1), lambda i: (i, 0)),
                   pl.BlockSpec((BM, 1), lambda i: (i, 0)),
                   pl.BlockSpec((BM, 1), lambda i: (i, 0)),
                   pl.BlockSpec((BM, 8), lambda i: (i, 0))],
        out_shape=[jax.ShapeDtypeStruct((M, 1), jnp.int32),
                   jax.ShapeDtypeStruct((M, 1), jnp.int32),
                   jax.ShapeDtypeStruct((M, 1), jnp.int32),
                   jax.ShapeDtypeStruct((M, 8), jnp.float32)],
    )(aq, asT, sn)


_SC_CH = 128


def _gather3(x, idx0, idx1, idx2):
    info = plsc.get_sparse_core_info()
    nw = info.num_cores * info.num_subcores
    b_per_w = M // nw
    n_ch = b_per_w // _SC_CH
    mesh = plsc.VectorSubcoreMesh(core_axis_name="c", subcore_axis_name="s")

    @functools.partial(
        pl.kernel, mesh=mesh,
        out_type=[jax.ShapeDtypeStruct((M, C_IN), jnp.float32)] * K,
        scratch_types=[
            pltpu.VMEM((b_per_w,), jnp.int32),
            pltpu.VMEM((b_per_w,), jnp.int32),
            pltpu.VMEM((b_per_w,), jnp.int32),
            pltpu.VMEM((_SC_CH, C_IN), jnp.float32),
            pltpu.VMEM((_SC_CH, C_IN), jnp.float32),
            pltpu.SemaphoreType.DMA,
            pltpu.SemaphoreType.DMA,
            pltpu.SemaphoreType.DMA,
            pltpu.SemaphoreType.DMA,
        ],
    )
    def k(x_hbm, i0_hbm, i1_hbm, i2_hbm, g0_hbm, g1_hbm, g2_hbm,
          i0_v, i1_v, i2_v, rows_a, rows_b, sga, sgb, ssa, ssb):
        wid = lax.axis_index("s") * info.num_cores + lax.axis_index("c")
        wbase = wid * b_per_w
        pltpu.sync_copy(i0_hbm.at[pl.ds(wbase, b_per_w)], i0_v)
        pltpu.sync_copy(i1_hbm.at[pl.ds(wbase, b_per_w)], i1_v)
        pltpu.sync_copy(i2_hbm.at[pl.ds(wbase, b_per_w)], i2_v)
        idxs = (i0_v, i1_v, i2_v)
        outs = (g0_hbm, g1_hbm, g2_hbm)
        steps = [(j, c) for j in range(K) for c in range(n_ch)]
        bufs = (rows_a, rows_b)
        gsems = (sga, sgb)
        ssems = (ssa, ssb)
        nst = len(steps)
        gh = [None] * nst
        sh = [None] * nst

        def scatter(t):
            j, c = steps[t]
            gh[t].wait()
            sh[t] = pltpu.async_copy(
                bufs[t % 2], outs[j].at[pl.ds(wbase + c * _SC_CH, _SC_CH)],
                ssems[t % 2])

        for t, (j, c) in enumerate(steps):
            if t >= 2:
                sh[t - 2].wait()
            gh[t] = pltpu.async_copy(
                x_hbm.at[idxs[j].at[pl.ds(c * _SC_CH, _SC_CH)]],
                bufs[t % 2], gsems[t % 2])
            if t >= 1:
                scatter(t - 1)
        scatter(nst - 1)
        sh[nst - 2].wait()
        sh[nst - 1].wait()

    return jax.jit(k)(x, idx0, idx1, idx2)


def _mlp_body(g0_ref, g1_ref, g2_ref, w_ref, xs_ref, w1a_ref, w1b_ref,
              w2_ref, b1_ref, b2_ref, gam_ref, bet_ref, gm_ref, gmT_ref,
              out_ref):
    w = w_ref[...]
    interp = (w[:, 0:1] * g0_ref[...] + w[:, 1:2] * g1_ref[...]
              + w[:, 2:3] * g2_ref[...])

    h = (_dot(interp, w1a_ref[...], precision=_DEF)
         + _dot(xs_ref[...], w1b_ref[...], precision=_DEF))
    h = jnp.maximum(h + b1_ref[...], 0.0)
    h = jnp.maximum(_dot(h, w2_ref[...], precision=_DEF) + b2_ref[...], 0.0)

    gm = gm_ref[...]
    gmT = gmT_ref[...]
    mean = _dot2(h, gm) * (1.0 / GSIZE)
    msq = _dot2(h * h, gm) * (1.0 / GSIZE)
    var = msq - mean * mean
    inv = jax.lax.rsqrt(var + EPS)
    hn = (h - _dot2(mean, gmT)) * _dot2(inv, gmT)
    out_ref[...] = hn * gam_ref[...] + bet_ref[...]


@jax.jit
def _mlp(g0, g1, g2, w8, x_skip, w1a, w1b, W2, b1, b2, gamma, beta, gm, gmT):
    full = lambda r, c: pl.BlockSpec((r, c), lambda i: (0, 0))
    return pl.pallas_call(
        _mlp_body,
        grid=(M // BM2,),
        in_specs=[
            pl.BlockSpec((BM2, C_IN), lambda i: (i, 0)),
            pl.BlockSpec((BM2, C_IN), lambda i: (i, 0)),
            pl.BlockSpec((BM2, C_IN), lambda i: (i, 0)),
            pl.BlockSpec((BM2, 8), lambda i: (i, 0)),
            pl.BlockSpec((BM2, C_SKIP), lambda i: (i, 0)),
            full(C_IN, C2), full(C_SKIP, C2), full(C2, C2),
            full(1, C2), full(1, C2), full(1, C2), full(1, C2),
            full(C2, GROUPS), full(GROUPS, C2),
        ],
        out_specs=pl.BlockSpec((BM2, C2), lambda i: (i, 0)),
        out_shape=jax.ShapeDtypeStruct((M, C2), jnp.float32),
    )(g0, g1, g2, w8, x_skip, w1a, w1b, W2, b1, b2, gamma, beta, gm, gmT)


def kernel(x, pos, reflectance, batch, x_skip, pos_skip, reflectance_skip,
           batch_skip, W1, b1, W2, b2, gamma, beta):
    qn = jnp.sum(pos_skip ** 2, axis=-1, keepdims=True)
    sn = jnp.sum(pos ** 2, axis=-1)[None, :]
    aq = jnp.concatenate([-2.0 * pos_skip, qn,
                          jnp.zeros((M, 4), jnp.float32)], axis=1)
    asT = jnp.concatenate([pos, jnp.zeros((N, 5), jnp.float32)], axis=1).T

    i0, i1, i2, w8 = _top3(aq, asT, sn)
    g0, g1, g2 = _gather3(x, i0.reshape(M), i1.reshape(M), i2.reshape(M))

    w1a = W1[:C_IN]
    w1b = W1[C_IN:]
    gidx = jnp.arange(C2, dtype=jnp.int32) // GSIZE
    gm = (gidx[:, None] == jnp.arange(GROUPS, dtype=jnp.int32)[None, :]).astype(jnp.float32)
    gmT = gm.T

    h = _mlp(g0, g1, g2, w8, x_skip, w1a, w1b, W2,
             b1.reshape(1, C2), b2.reshape(1, C2),
             gamma.reshape(1, C2), beta.reshape(1, C2), gm, gmT)
    return (h, pos_skip, reflectance_skip, batch_skip)
